# baseline (speedup 1.0000x reference)
import jax
import jax.numpy as jnp
from jax.experimental import pallas as pl
from jax.experimental.pallas import tpu as pltpu

L1_CAP, L2_CAP, L3_CAP = 2048, 4096, 8192
SDR, CDIM = 2048, 384
N = 2048


def _dma_body(sdrs, contents, l1s, l1c, l2s, l2c, l3s, l3c,
              o1s, o1c, o2s, o2c, o3s, o3c, sem):
    h = N
    plan = [
        (sdrs, o1s),
        (contents, o1c),
        (l1s, o2s.at[pl.ds(0, h)]),
        (l1c, o2c.at[pl.ds(0, h)]),
        (l2s.at[pl.ds(h, L2_CAP - h)], o2s.at[pl.ds(h, L2_CAP - h)]),
        (l2c.at[pl.ds(h, L2_CAP - h)], o2c.at[pl.ds(h, L2_CAP - h)]),
        (l2s.at[pl.ds(0, h)], o3s.at[pl.ds(0, h)]),
        (l2c.at[pl.ds(0, h)], o3c.at[pl.ds(0, h)]),
        (l3s.at[pl.ds(h, L3_CAP - h)], o3s.at[pl.ds(h, L3_CAP - h)]),
        (l3c.at[pl.ds(h, L3_CAP - h)], o3c.at[pl.ds(h, L3_CAP - h)]),
    ]
    copies = [pltpu.make_async_copy(src, dst, sem.at[i])
              for i, (src, dst) in enumerate(plan)]
    for c in copies:
        c.start()
    for c in copies:
        c.wait()


def kernel(sdrs, contents, l1_sdr_bank, l1_content_bank,
           l2_sdr_bank, l2_content_bank, l3_sdr_bank, l3_content_bank):
    sdrs = jax.lax.stop_gradient(sdrs)
    contents = jax.lax.stop_gradient(contents)
    out_shape = [
        jax.ShapeDtypeStruct((L1_CAP, SDR), jnp.float32),
        jax.ShapeDtypeStruct((L1_CAP, CDIM), jnp.float32),
        jax.ShapeDtypeStruct((L2_CAP, SDR), jnp.float32),
        jax.ShapeDtypeStruct((L2_CAP, CDIM), jnp.float32),
        jax.ShapeDtypeStruct((L3_CAP, SDR), jnp.float32),
        jax.ShapeDtypeStruct((L3_CAP, CDIM), jnp.float32),
    ]
    any_spec = pl.BlockSpec(memory_space=pl.ANY)
    outs = pl.pallas_call(
        _dma_body,
        out_shape=out_shape,
        in_specs=[any_spec] * 8,
        out_specs=[any_spec] * 6,
        scratch_shapes=[pltpu.SemaphoreType.DMA((10,))],
    )(sdrs, contents, l1_sdr_bank, l1_content_bank,
      l2_sdr_bank, l2_content_bank, l3_sdr_bank, l3_content_bank)
    return tuple(outs)
